# baseline (device time: 21829 ns/iter reference)
import jax
import jax.numpy as jnp
from jax import lax
from jax.experimental import pallas as pl
from jax.experimental.pallas import tpu as pltpu

RED_CHUNKS = 8
RED_ROWS = 120
TAIL = 64


def kernel(x):
    m, n = x.shape
    assert RED_CHUNKS * RED_ROWS + TAIL == m

    def body(
        x_ref, out_ref, xb_ref, rx_ref, rawy_ref, rawd_ref,
        sx_sems, rxv_sems, sy_sems, ry_sems, sraw_sems, rraw_sems,
    ):
        my_x = lax.axis_index("x")
        my_y = lax.axis_index("y")
        x_nbr = (1 - my_x, my_y)
        y_nbr = (my_x, 1 - my_y)
        diag = (1 - my_x, 1 - my_y)

        barrier_sem = pltpu.get_barrier_semaphore()
        for nbr in (x_nbr, y_nbr, diag):
            pl.semaphore_signal(
                barrier_sem, inc=1,
                device_id=nbr, device_id_type=pl.DeviceIdType.MESH,
            )
        pl.semaphore_wait(barrier_sem, 3)

        rows = lambda k: pl.ds(k * RED_ROWS, RED_ROWS)
        tail = pl.ds(RED_CHUNKS * RED_ROWS, TAIL)
        my_col = pl.ds(my_y * n, n)
        other_col = pl.ds((1 - my_y) * n, n)

        xb_ref[tail, :] = x_ref[tail, :].astype(jnp.bfloat16)
        raw_rdmas = []
        for i, (dev, dst) in enumerate(((y_nbr, rawy_ref), (diag, rawd_ref))):
            r = pltpu.make_async_remote_copy(
                src_ref=xb_ref.at[tail, :],
                dst_ref=dst,
                send_sem=sraw_sems.at[i],
                recv_sem=rraw_sems.at[i],
                device_id=dev,
                device_id_type=pl.DeviceIdType.MESH,
            )
            r.start()
            raw_rdmas.append(r)

        rdmas_x = []
        for k in range(RED_CHUNKS + 1):
            src = tail if k == RED_CHUNKS else rows(k)
            if k < RED_CHUNKS:
                xb_ref[src, :] = x_ref[src, :].astype(jnp.bfloat16)
            r = pltpu.make_async_remote_copy(
                src_ref=xb_ref.at[src, :],
                dst_ref=rx_ref.at[src, :],
                send_sem=sx_sems.at[k],
                recv_sem=rxv_sems.at[k],
                device_id=x_nbr,
                device_id_type=pl.DeviceIdType.MESH,
            )
            r.start()
            rdmas_x.append(r)

        rdmas_y = []
        for k in range(RED_CHUNKS):
            rdmas_x[k].wait_recv()
            out_ref[rows(k), my_col] = xb_ref[rows(k), :] + rx_ref[rows(k), :]
            r = pltpu.make_async_remote_copy(
                src_ref=out_ref.at[rows(k), my_col],
                dst_ref=out_ref.at[rows(k), my_col],
                send_sem=sy_sems.at[k],
                recv_sem=ry_sems.at[k],
                device_id=y_nbr,
                device_id_type=pl.DeviceIdType.MESH,
            )
            r.start()
            rdmas_y.append(r)

        rdmas_x[RED_CHUNKS].wait_recv()
        out_ref[tail, my_col] = xb_ref[tail, :] + rx_ref[tail, :]
        raw_rdmas[0].wait_recv()
        raw_rdmas[1].wait_recv()
        out_ref[tail, other_col] = rawy_ref[...] + rawd_ref[...]

        for k in range(RED_CHUNKS):
            rdmas_y[k].wait_recv()
        for r in rdmas_x + rdmas_y + raw_rdmas:
            r.wait_send()

    return pl.pallas_call(
        body,
        out_shape=jax.ShapeDtypeStruct((m, 2 * n), jnp.bfloat16),
        in_specs=[pl.BlockSpec(memory_space=pltpu.VMEM)],
        out_specs=pl.BlockSpec(memory_space=pltpu.VMEM),
        scratch_shapes=[
            pltpu.VMEM((m, n), jnp.bfloat16),
            pltpu.VMEM((m, n), jnp.bfloat16),
            pltpu.VMEM((TAIL, n), jnp.bfloat16),
            pltpu.VMEM((TAIL, n), jnp.bfloat16),
            pltpu.SemaphoreType.DMA((RED_CHUNKS + 1,)),
            pltpu.SemaphoreType.DMA((RED_CHUNKS + 1,)),
            pltpu.SemaphoreType.DMA((RED_CHUNKS,)),
            pltpu.SemaphoreType.DMA((RED_CHUNKS,)),
            pltpu.SemaphoreType.DMA((2,)),
            pltpu.SemaphoreType.DMA((2,)),
        ],
        compiler_params=pltpu.CompilerParams(collective_id=0),
    )(x)
